# baseline (speedup 1.0000x reference)
import functools

import jax
import jax.numpy as jnp
from jax.experimental import pallas as pl
from jax.experimental.pallas import tpu as pltpu


def _body(J, Fin, H, C, EP,
          x_ref, p_ref, wfc_ref, o_ref, g1_scr, g2_scr, bt_scr, wfc_scr):
    KC = J * Fin
    JH = J * H
    f32 = jnp.float32
    R_W1, R_W2 = 8, 16
    R_DC = R_W2 + H

    @pl.when(pl.program_id(0) == 0)
    def _init():
        src = p_ref[0:1, :]
        dst = p_ref[1:2, :]
        dst_col = p_ref[R_DC:R_DC + EP, 0:1]
        iota_j = jax.lax.broadcasted_iota(jnp.int32, (J, EP), 0).astype(f32)
        od = (dst == iota_j).astype(f32)
        osrc = (src == iota_j).astype(f32)
        deg = 1.0 + jnp.sum(od, axis=1, keepdims=True)
        dinv = jax.lax.rsqrt(deg)
        wgt = (jnp.sum(od * dinv, axis=0, keepdims=True)
               * jnp.sum(osrc * dinv, axis=0, keepdims=True))
        iota_jt = jax.lax.broadcasted_iota(jnp.int32, (EP, J), 1).astype(f32)
        odT = (dst_col == iota_jt).astype(f32)
        eye = (jax.lax.broadcasted_iota(jnp.int32, (J, J), 0)
               == jax.lax.broadcasted_iota(jnp.int32, (J, J), 1)).astype(f32)
        At = (jnp.dot(osrc * wgt, odT, preferred_element_type=f32)
              + eye * (1.0 / deg))

        def expand(shape, blk, mode):
            a0 = jax.lax.broadcasted_iota(jnp.int32, shape, 0)
            a1 = jax.lax.broadcasted_iota(jnp.int32, shape, 1)
            if mode == "div0":
                return (a0 // blk == a1).astype(f32)
            if mode == "mod0":
                return (a0 % blk == a1).astype(f32)
            return (a1 % blk == a0).astype(f32)

        EjT = (jax.lax.broadcasted_iota(jnp.int32, (J, JH), 1) // H
               == jax.lax.broadcasted_iota(jnp.int32, (J, JH), 0)).astype(f32)
        AtE = jnp.dot(At, EjT, preferred_element_type=f32)
        Ftile = expand((H, JH), H, "mod1")

        Ek1 = expand((KC, J), Fin, "div0")
        Rc1 = expand((KC, 8), Fin, "mod0")
        PW1 = jnp.dot(jnp.dot(Rc1, p_ref[R_W1:R_W1 + 8, :H],
                              preferred_element_type=f32),
                      Ftile, preferred_element_type=f32)
        g1_scr[...] = (jnp.dot(Ek1, AtE, preferred_element_type=f32)
                       * PW1).astype(jnp.bfloat16)

        Ek2 = expand((JH, J), H, "div0")
        Rf2 = expand((JH, H), H, "mod0")
        PW2 = jnp.dot(jnp.dot(Rf2, p_ref[R_W2:R_W2 + H, :H],
                              preferred_element_type=f32),
                      Ftile, preferred_element_type=f32)
        g2_scr[...] = (jnp.dot(Ek2, AtE, preferred_element_type=f32)
                       * PW2).astype(jnp.bfloat16)

        bt_scr[0:1, :] = jnp.dot(p_ref[2:3, :H], Ftile,
                                 preferred_element_type=f32)
        bt_scr[1:2, :] = jnp.dot(p_ref[3:4, :H], Ftile,
                                 preferred_element_type=f32)
        wfc_scr[...] = wfc_ref[...].astype(jnp.bfloat16)

    xb = x_ref[...].astype(jnp.bfloat16)
    h1 = jnp.maximum(jnp.dot(xb, g1_scr[...],
                             preferred_element_type=f32).astype(jnp.bfloat16)
                     + bt_scr[0:1, :].astype(jnp.bfloat16), 0.0)
    h2 = jnp.maximum(jnp.dot(h1, g2_scr[...],
                             preferred_element_type=f32).astype(jnp.bfloat16)
                     + bt_scr[1:2, :].astype(jnp.bfloat16), 0.0)
    o_ref[...] = (jnp.dot(h2, wfc_scr[...], preferred_element_type=f32)
                  + p_ref[4:5, :C])


@jax.jit
def kernel(x, edge_index, W1, b1, W2, b2, Wfc, bfc):
    B, J, Fin = x.shape
    H = W1.shape[1]
    C = Wfc.shape[1]
    E = edge_index.shape[1]
    KC = J * Fin
    JH = J * H
    f32 = jnp.float32

    x_flat = x.reshape(B, KC)

    EP = max(64, ((E + 63) // 64) * 64)
    W = max(EP, H)
    srow = jnp.concatenate([edge_index[0].astype(f32),
                            jnp.full((W - E,), float(J), f32)])
    drow = jnp.concatenate([edge_index[1].astype(f32),
                            jnp.full((W - E,), float(J), f32)])
    params = jnp.concatenate([
        srow[None], drow[None],
        jnp.pad(b1, (0, W - H))[None],
        jnp.pad(b2, (0, W - H))[None],
        jnp.pad(bfc, (0, W - C))[None],
        jnp.zeros((3, W), f32),
        jnp.pad(W1, ((0, 8 - Fin), (0, W - H))),
        jnp.pad(W2, ((0, 0), (0, W - H))),
        jnp.full((EP, W), float(J), f32).at[:, 0].set(drow[:EP]),
    ], axis=0)

    TB = 4096
    while B % TB:
        TB //= 2
    grid = (B // TB,)

    out = pl.pallas_call(
        functools.partial(_body, J, Fin, H, C, EP),
        grid=grid,
        in_specs=[
            pl.BlockSpec((TB, KC), lambda i: (i, 0)),
            pl.BlockSpec(params.shape, lambda i: (0, 0)),
            pl.BlockSpec((JH, C), lambda i: (0, 0)),
        ],
        out_specs=pl.BlockSpec((TB, C), lambda i: (i, 0)),
        out_shape=jax.ShapeDtypeStruct((B, C), x.dtype),
        scratch_shapes=[
            pltpu.VMEM((KC, JH), jnp.bfloat16),
            pltpu.VMEM((JH, JH), jnp.bfloat16),
            pltpu.VMEM((8, JH), jnp.float32),
            pltpu.VMEM((JH, C), jnp.bfloat16),
        ],
        compiler_params=pltpu.CompilerParams(
            dimension_semantics=("arbitrary",),
        ),
    )(x_flat, params, Wfc)
    return out


# speedup vs baseline: 1.0062x; 1.0062x over previous
import functools

import jax
import jax.numpy as jnp
from jax.experimental import pallas as pl
from jax.experimental.pallas import tpu as pltpu


def _body(J, Fin, H, C, EP,
          x_ref, p_ref, wfc_ref, o_ref, g1_scr, g2_scr, bt_scr, wfc_scr):
    KC = J * Fin
    JH = J * H
    f32 = jnp.float32
    R_W1, R_W2 = 8, 16
    R_DC = R_W2 + H

    @pl.when(pl.program_id(0) == 0)
    def _init():
        src = p_ref[0:1, :]
        dst = p_ref[1:2, :]
        dst_col = p_ref[R_DC:R_DC + EP, 0:1]
        iota_j = jax.lax.broadcasted_iota(jnp.int32, (J, EP), 0).astype(f32)
        od = (dst == iota_j).astype(f32)
        osrc = (src == iota_j).astype(f32)
        deg = 1.0 + jnp.sum(od, axis=1, keepdims=True)
        dinv = jax.lax.rsqrt(deg)
        wgt = (jnp.sum(od * dinv, axis=0, keepdims=True)
               * jnp.sum(osrc * dinv, axis=0, keepdims=True))
        iota_jt = jax.lax.broadcasted_iota(jnp.int32, (EP, J), 1).astype(f32)
        odT = (dst_col == iota_jt).astype(f32)
        eye = (jax.lax.broadcasted_iota(jnp.int32, (J, J), 0)
               == jax.lax.broadcasted_iota(jnp.int32, (J, J), 1)).astype(f32)
        At = (jnp.dot(osrc * wgt, odT, preferred_element_type=f32)
              + eye * (1.0 / deg))

        def expand(shape, blk, mode):
            a0 = jax.lax.broadcasted_iota(jnp.int32, shape, 0)
            a1 = jax.lax.broadcasted_iota(jnp.int32, shape, 1)
            if mode == "div0":
                return (a0 // blk == a1).astype(f32)
            if mode == "mod0":
                return (a0 % blk == a1).astype(f32)
            return (a1 % blk == a0).astype(f32)

        EjT = (jax.lax.broadcasted_iota(jnp.int32, (J, JH), 1) // H
               == jax.lax.broadcasted_iota(jnp.int32, (J, JH), 0)).astype(f32)
        AtE = jnp.dot(At, EjT, preferred_element_type=f32)
        Ftile = expand((H, JH), H, "mod1")

        Ek1 = expand((KC, J), Fin, "div0")
        Rc1 = expand((KC, 8), Fin, "mod0")
        PW1 = jnp.dot(jnp.dot(Rc1, p_ref[R_W1:R_W1 + 8, :H],
                              preferred_element_type=f32),
                      Ftile, preferred_element_type=f32)
        g1_scr[...] = (jnp.dot(Ek1, AtE, preferred_element_type=f32)
                       * PW1).astype(jnp.bfloat16)

        Ek2 = expand((JH, J), H, "div0")
        Rf2 = expand((JH, H), H, "mod0")
        PW2 = jnp.dot(jnp.dot(Rf2, p_ref[R_W2:R_W2 + H, :H],
                              preferred_element_type=f32),
                      Ftile, preferred_element_type=f32)
        g2_scr[...] = (jnp.dot(Ek2, AtE, preferred_element_type=f32)
                       * PW2).astype(jnp.bfloat16)

        bt_scr[0:1, :] = jnp.dot(p_ref[2:3, :H], Ftile,
                                 preferred_element_type=f32)
        bt_scr[1:2, :] = jnp.dot(p_ref[3:4, :H], Ftile,
                                 preferred_element_type=f32)
        wfc_scr[...] = wfc_ref[...].astype(jnp.bfloat16)

    TBH = x_ref.shape[0] // 2
    for half in range(2):
        rows = pl.ds(half * TBH, TBH)
        xb = x_ref[rows, :].astype(jnp.bfloat16)
        h1 = jnp.maximum(jnp.dot(xb, g1_scr[...],
                                 preferred_element_type=f32)
                         .astype(jnp.bfloat16)
                         + bt_scr[0:1, :].astype(jnp.bfloat16), 0.0)
        h2 = jnp.maximum(jnp.dot(h1, g2_scr[...],
                                 preferred_element_type=f32)
                         .astype(jnp.bfloat16)
                         + bt_scr[1:2, :].astype(jnp.bfloat16), 0.0)
        o_ref[rows, :] = (jnp.dot(h2, wfc_scr[...],
                                  preferred_element_type=f32)
                          + p_ref[4:5, :C])


@jax.jit
def kernel(x, edge_index, W1, b1, W2, b2, Wfc, bfc):
    B, J, Fin = x.shape
    H = W1.shape[1]
    C = Wfc.shape[1]
    E = edge_index.shape[1]
    KC = J * Fin
    JH = J * H
    f32 = jnp.float32

    x_flat = x.reshape(B, KC)

    EP = max(64, ((E + 63) // 64) * 64)
    W = max(EP, H)
    srow = jnp.concatenate([edge_index[0].astype(f32),
                            jnp.full((W - E,), float(J), f32)])
    drow = jnp.concatenate([edge_index[1].astype(f32),
                            jnp.full((W - E,), float(J), f32)])
    params = jnp.concatenate([
        srow[None], drow[None],
        jnp.pad(b1, (0, W - H))[None],
        jnp.pad(b2, (0, W - H))[None],
        jnp.pad(bfc, (0, W - C))[None],
        jnp.zeros((3, W), f32),
        jnp.pad(W1, ((0, 8 - Fin), (0, W - H))),
        jnp.pad(W2, ((0, 0), (0, W - H))),
        jnp.full((EP, W), float(J), f32).at[:, 0].set(drow[:EP]),
    ], axis=0)

    TB = 4096
    while B % TB:
        TB //= 2
    grid = (B // TB,)

    out = pl.pallas_call(
        functools.partial(_body, J, Fin, H, C, EP),
        grid=grid,
        in_specs=[
            pl.BlockSpec((TB, KC), lambda i: (i, 0)),
            pl.BlockSpec(params.shape, lambda i: (0, 0)),
            pl.BlockSpec((JH, C), lambda i: (0, 0)),
        ],
        out_specs=pl.BlockSpec((TB, C), lambda i: (i, 0)),
        out_shape=jax.ShapeDtypeStruct((B, C), x.dtype),
        scratch_shapes=[
            pltpu.VMEM((KC, JH), jnp.bfloat16),
            pltpu.VMEM((JH, JH), jnp.bfloat16),
            pltpu.VMEM((8, JH), jnp.float32),
            pltpu.VMEM((JH, C), jnp.bfloat16),
        ],
        compiler_params=pltpu.CompilerParams(
            dimension_semantics=("arbitrary",),
        ),
    )(x_flat, params, Wfc)
    return out


# speedup vs baseline: 1.0065x; 1.0003x over previous
import functools

import jax
import jax.numpy as jnp
from jax.experimental import pallas as pl
from jax.experimental.pallas import tpu as pltpu


def _body(J, Fin, H, C, EP,
          x_ref, p_ref, wfc_ref, o_ref, g1_scr, g2_scr, bt_scr, wfc_scr):
    KC = J * Fin
    JH = J * H
    f32 = jnp.float32
    R_W1, R_W2 = 8, 16
    R_DC = R_W2 + H

    @pl.when(pl.program_id(0) == 0)
    def _init():
        src = p_ref[0:1, :]
        dst = p_ref[1:2, :]
        dst_col = p_ref[R_DC:R_DC + EP, 0:1]
        iota_j = jax.lax.broadcasted_iota(jnp.int32, (J, EP), 0).astype(f32)
        od = (dst == iota_j).astype(f32)
        osrc = (src == iota_j).astype(f32)
        deg = 1.0 + jnp.sum(od, axis=1, keepdims=True)
        dinv = jax.lax.rsqrt(deg)
        wgt = (jnp.sum(od * dinv, axis=0, keepdims=True)
               * jnp.sum(osrc * dinv, axis=0, keepdims=True))
        iota_jt = jax.lax.broadcasted_iota(jnp.int32, (EP, J), 1).astype(f32)
        odT = (dst_col == iota_jt).astype(f32)
        eye = (jax.lax.broadcasted_iota(jnp.int32, (J, J), 0)
               == jax.lax.broadcasted_iota(jnp.int32, (J, J), 1)).astype(f32)
        At = (jnp.dot(osrc * wgt, odT, preferred_element_type=f32)
              + eye * (1.0 / deg))

        def expand(shape, blk, mode):
            a0 = jax.lax.broadcasted_iota(jnp.int32, shape, 0)
            a1 = jax.lax.broadcasted_iota(jnp.int32, shape, 1)
            if mode == "div0":
                return (a0 // blk == a1).astype(f32)
            if mode == "mod0":
                return (a0 % blk == a1).astype(f32)
            return (a1 % blk == a0).astype(f32)

        EjT = (jax.lax.broadcasted_iota(jnp.int32, (J, JH), 1) // H
               == jax.lax.broadcasted_iota(jnp.int32, (J, JH), 0)).astype(f32)
        AtE = jnp.dot(At, EjT, preferred_element_type=f32)
        Ftile = expand((H, JH), H, "mod1")

        Ek1 = expand((KC, J), Fin, "div0")
        Rc1 = expand((KC, 8), Fin, "mod0")
        PW1 = jnp.dot(jnp.dot(Rc1, p_ref[R_W1:R_W1 + 8, :H],
                              preferred_element_type=f32),
                      Ftile, preferred_element_type=f32)
        g1_scr[...] = (jnp.dot(Ek1, AtE, preferred_element_type=f32)
                       * PW1).astype(jnp.bfloat16)

        Ek2 = expand((JH, J), H, "div0")
        Rf2 = expand((JH, H), H, "mod0")
        PW2 = jnp.dot(jnp.dot(Rf2, p_ref[R_W2:R_W2 + H, :H],
                              preferred_element_type=f32),
                      Ftile, preferred_element_type=f32)
        g2_scr[...] = (jnp.dot(Ek2, AtE, preferred_element_type=f32)
                       * PW2).astype(jnp.bfloat16)

        bt_scr[0:1, :] = jnp.dot(p_ref[2:3, :H], Ftile,
                                 preferred_element_type=f32)
        bt_scr[1:2, :] = jnp.dot(p_ref[3:4, :H], Ftile,
                                 preferred_element_type=f32)
        wfc_scr[...] = wfc_ref[...].astype(jnp.bfloat16)

    TBH = x_ref.shape[0] // 4
    for half in range(4):
        rows = pl.ds(half * TBH, TBH)
        xb = x_ref[rows, :].astype(jnp.bfloat16)
        h1 = jnp.maximum(jnp.dot(xb, g1_scr[...],
                                 preferred_element_type=f32)
                         .astype(jnp.bfloat16)
                         + bt_scr[0:1, :].astype(jnp.bfloat16), 0.0)
        h2 = jnp.maximum(jnp.dot(h1, g2_scr[...],
                                 preferred_element_type=f32)
                         .astype(jnp.bfloat16)
                         + bt_scr[1:2, :].astype(jnp.bfloat16), 0.0)
        o_ref[rows, :] = (jnp.dot(h2, wfc_scr[...],
                                  preferred_element_type=f32)
                          + p_ref[4:5, :C])


@jax.jit
def kernel(x, edge_index, W1, b1, W2, b2, Wfc, bfc):
    B, J, Fin = x.shape
    H = W1.shape[1]
    C = Wfc.shape[1]
    E = edge_index.shape[1]
    KC = J * Fin
    JH = J * H
    f32 = jnp.float32

    x_flat = x.reshape(B, KC)

    EP = max(64, ((E + 63) // 64) * 64)
    W = max(EP, H)
    srow = jnp.concatenate([edge_index[0].astype(f32),
                            jnp.full((W - E,), float(J), f32)])
    drow = jnp.concatenate([edge_index[1].astype(f32),
                            jnp.full((W - E,), float(J), f32)])
    params = jnp.concatenate([
        srow[None], drow[None],
        jnp.pad(b1, (0, W - H))[None],
        jnp.pad(b2, (0, W - H))[None],
        jnp.pad(bfc, (0, W - C))[None],
        jnp.zeros((3, W), f32),
        jnp.pad(W1, ((0, 8 - Fin), (0, W - H))),
        jnp.pad(W2, ((0, 0), (0, W - H))),
        jnp.full((EP, W), float(J), f32).at[:, 0].set(drow[:EP]),
    ], axis=0)

    TB = 4096
    while B % TB:
        TB //= 2
    grid = (B // TB,)

    out = pl.pallas_call(
        functools.partial(_body, J, Fin, H, C, EP),
        grid=grid,
        in_specs=[
            pl.BlockSpec((TB, KC), lambda i: (i, 0)),
            pl.BlockSpec(params.shape, lambda i: (0, 0)),
            pl.BlockSpec((JH, C), lambda i: (0, 0)),
        ],
        out_specs=pl.BlockSpec((TB, C), lambda i: (i, 0)),
        out_shape=jax.ShapeDtypeStruct((B, C), x.dtype),
        scratch_shapes=[
            pltpu.VMEM((KC, JH), jnp.bfloat16),
            pltpu.VMEM((JH, JH), jnp.bfloat16),
            pltpu.VMEM((8, JH), jnp.float32),
            pltpu.VMEM((JH, C), jnp.bfloat16),
        ],
        compiler_params=pltpu.CompilerParams(
            dimension_semantics=("arbitrary",),
        ),
    )(x_flat, params, Wfc)
    return out
